# baseline (speedup 1.0000x reference)
import functools

import jax
import jax.numpy as jnp
from jax import lax
from jax.experimental import pallas as pl
from jax.experimental.pallas import tpu as pltpu
from jax.experimental.pallas import tpu_sc as plsc

N = 10000
D = 128
E = 160000
H = 64

NC = 2
NS = 16
C = 128
NCHUNK = 80
EPT = NCHUNK * C
PAD_E = EPT * NS
ACC_N = 10112
ROWS_PT = ACC_N // NS


@functools.cache
def _sc_agg():
  def body(table, idx5, zfeat, out_sums, acc, idx0, idx1, rows0, rows1,
           gsem0, gsem1):
    c = lax.axis_index("c")
    s = lax.axis_index("s")
    idx = (idx0, idx1)
    rows = (rows0, rows1)
    gsem = (gsem0, gsem1)

    r0 = pl.multiple_of(s * ROWS_PT, 8)
    pltpu.sync_copy(zfeat.at[pl.ds(r0, ROWS_PT)], acc.at[pl.ds(r0, ROWS_PT)])
    plsc.subcore_barrier()

    pltpu.sync_copy(idx5.at[c, s, 0], idx[0])
    pltpu.sync_copy(table.at[idx[0].at[0]], rows[0])

    def outer(j2, carry):
      for b in range(2):
        jj = j2 * 2 + b
        nb = 1 - b

        @pl.when(jj + 1 < NCHUNK)
        def _overlapped():
          pltpu.sync_copy(idx5.at[c, s, jj + 1], idx[nb])
          cp = pltpu.async_copy(table.at[idx[nb].at[0]], rows[nb], gsem[nb])
          pltpu.sync_copy(rows[b], acc.at[idx[b].at[1]], add=True)
          cp.wait()

        @pl.when(jj + 1 >= NCHUNK)
        def _last_chunk():
          pltpu.sync_copy(rows[b], acc.at[idx[b].at[1]], add=True)
      return carry

    lax.fori_loop(0, NCHUNK // 2, outer, 0)
    plsc.subcore_barrier()

    pltpu.sync_copy(acc.at[pl.ds(r0, ROWS_PT)], out_sums.at[c, pl.ds(r0, ROWS_PT)])

  mesh = plsc.VectorSubcoreMesh(core_axis_name="c", subcore_axis_name="s")
  return pl.kernel(
      body,
      out_type=[jax.ShapeDtypeStruct((NC, ACC_N, D), jnp.float32)],
      mesh=mesh,
      scratch_types=[
          pltpu.VMEM_SHARED((ACC_N, D), jnp.float32),
          pltpu.VMEM((2, C), jnp.int32),
          pltpu.VMEM((2, C), jnp.int32),
          pltpu.VMEM((C, D), jnp.float32),
          pltpu.VMEM((C, D), jnp.float32),
          pltpu.SemaphoreType.DMA,
          pltpu.SemaphoreType.DMA,
      ])


@functools.cache
def _sc_deg():
  def body(idx5, zfeat, ones_c, out_deg, acc, idx0, ones_v):
    c = lax.axis_index("c")
    s = lax.axis_index("s")

    r0 = pl.multiple_of(s * ROWS_PT, 8)
    pltpu.sync_copy(zfeat.at[pl.ds(r0, ROWS_PT)], acc.at[pl.ds(r0, ROWS_PT)])
    pltpu.sync_copy(ones_c, ones_v)
    plsc.subcore_barrier()

    def chunk(ci, carry):
      pltpu.sync_copy(idx5.at[c, s, ci], idx0)
      pltpu.sync_copy(ones_v, acc.at[idx0.at[1]], add=True)
      return carry

    lax.fori_loop(0, NCHUNK, chunk, 0)
    plsc.subcore_barrier()

    pltpu.sync_copy(acc.at[pl.ds(r0, ROWS_PT)], out_deg.at[c, pl.ds(r0, ROWS_PT)])

  mesh = plsc.VectorSubcoreMesh(core_axis_name="c", subcore_axis_name="s")
  return pl.kernel(
      body,
      out_type=[jax.ShapeDtypeStruct((NC, ACC_N, D), jnp.float32)],
      mesh=mesh,
      scratch_types=[
          pltpu.VMEM_SHARED((ACC_N, D), jnp.float32),
          pltpu.VMEM((2, C), jnp.int32),
          pltpu.VMEM((C, D), jnp.float32),
      ])


_BN = 1000


def _tc_base_body(sp, sn, dp, dn, x, wp, bp, wn, bn, out):
  aggp = sp[...] / jnp.maximum(dp[:, 0:1], 1.0)
  aggn = sn[...] / jnp.maximum(dn[:, 0:1], 1.0)
  xb = x[...]
  out[:, 0:H] = jnp.tanh(aggp @ wp[0:D] + xb @ wp[D:2 * D] + bp[...])
  out[:, H:2 * H] = jnp.tanh(aggn @ wn[0:D] + xb @ wn[D:2 * D] + bn[...])


def _tc_deep_body(sp, sn, dp, dn, hcat, wp, bp, wn, bn, out):
  aggp = sp[...] / jnp.maximum(dp[:, 0:1], 1.0)
  aggn = sn[...] / jnp.maximum(dn[:, 0:1], 1.0)
  hb = hcat[...]
  out[:, 0:H] = jnp.tanh(
      aggp[:, 0:H] @ wp[0:H] + aggn[:, H:2 * H] @ wp[H:2 * H]
      + hb[:, 0:H] @ wp[2 * H:3 * H] + bp[...])
  out[:, H:2 * H] = jnp.tanh(
      aggn[:, 0:H] @ wn[H:2 * H] + aggp[:, H:2 * H] @ wn[0:H]
      + hb[:, H:2 * H] @ wn[2 * H:3 * H] + bn[...])


def _row_block(feat):
  return pl.BlockSpec((_BN, feat), lambda i: (i, 0))


def _full_block(shape):
  return pl.BlockSpec(shape, lambda i: (0,) * len(shape))


def _make_tc(body, kdim):
  return pl.pallas_call(
      body,
      grid=(N // _BN,),
      in_specs=[
          _row_block(D), _row_block(D), _row_block(D), _row_block(D),
          _row_block(D),
          _full_block((kdim, H)), _full_block((1, H)),
          _full_block((kdim, H)), _full_block((1, H)),
      ],
      out_specs=_row_block(D),
      out_shape=jax.ShapeDtypeStruct((N, D), jnp.float32),
  )


_tc_base = _make_tc(_tc_base_body, 2 * D)
_tc_deep = _make_tc(_tc_deep_body, 3 * H)


def kernel(positive_edges, negative_edges, pos_adj, neg_adj, X,
           W_pos_base, b_pos_base, W_neg_base, b_neg_base,
           W_pos_deep, b_pos_deep, W_neg_deep, b_neg_deep):
  del pos_adj, neg_adj
  pad = PAD_E - E
  pad_src = jnp.zeros((1, pad), jnp.int32)
  pad_dst = jnp.full((1, pad), N, jnp.int32)
  srcs = jnp.concatenate([jnp.stack([positive_edges[0], negative_edges[0]]),
                          jnp.broadcast_to(pad_src, (NC, pad))], axis=1)
  dsts = jnp.concatenate([jnp.stack([positive_edges[1], negative_edges[1]]),
                          jnp.broadcast_to(pad_dst, (NC, pad))], axis=1)
  idx5 = jnp.stack([srcs.reshape(NC, NS, NCHUNK, C),
                    dsts.reshape(NC, NS, NCHUNK, C)], axis=3)

  zfeat = jnp.zeros((ACC_N, D), jnp.float32)
  ones_c = jnp.ones((C, D), jnp.float32)

  (deg,) = _sc_deg()(idx5, zfeat, ones_c)
  (sums1,) = _sc_agg()(X, idx5, zfeat)
  hcat = _tc_base(sums1[0], sums1[1], deg[0], deg[1], X,
                  W_pos_base, b_pos_base.reshape(1, H),
                  W_neg_base, b_neg_base.reshape(1, H))
  (sums2,) = _sc_agg()(hcat, idx5, zfeat)
  z = _tc_deep(sums2[0], sums2[1], deg[0], deg[1], hcat,
               W_pos_deep, b_pos_deep.reshape(1, H),
               W_neg_deep, b_neg_deep.reshape(1, H))
  return z


# speedup vs baseline: 1.0804x; 1.0804x over previous
import functools

import jax
import jax.numpy as jnp
from jax import lax
from jax.experimental import pallas as pl
from jax.experimental.pallas import tpu as pltpu
from jax.experimental.pallas import tpu_sc as plsc

N = 10000
D = 128
E = 160000
H = 64

NC = 2
NS = 16
C = 128
GB = 8
NCHUNK = 80
EPT = NCHUNK * C
PAD_E = EPT * NS
ACC_N = 10112
ROWS_PT = ACC_N // NS


@functools.cache
def _sc_agg():
  def body(table, idx5, zfeat, out_sums, acc, idxblk, rows0, rows1,
           gsem0, gsem1, ssem0, ssem1):
    c = lax.axis_index("c")
    s = lax.axis_index("s")
    rows = (rows0, rows1)
    gsem = (gsem0, gsem1)
    ssem = (ssem0, ssem1)

    r0 = pl.multiple_of(s * ROWS_PT, 8)
    pltpu.sync_copy(zfeat.at[pl.ds(r0, ROWS_PT)], acc.at[pl.ds(r0, ROWS_PT)])
    plsc.subcore_barrier()

    def group(gi, carry):
      pltpu.sync_copy(idx5.at[c, s, pl.ds(gi * GB, GB)], idxblk)
      g = [None] * GB
      sd = [None] * GB
      g[0] = pltpu.async_copy(table.at[idxblk.at[0, 0]], rows[0], gsem[0])
      g[1] = pltpu.async_copy(table.at[idxblk.at[1, 0]], rows[1], gsem[1])
      for k in range(GB):
        b = k & 1
        g[k].wait()
        sd[k] = pltpu.async_copy(rows[b], acc.at[idxblk.at[k, 1]], ssem[b],
                                 add=True)
        if k + 2 < GB:
          sd[k].wait()
          g[k + 2] = pltpu.async_copy(table.at[idxblk.at[k + 2, 0]], rows[b],
                                      gsem[b])
      sd[GB - 2].wait()
      sd[GB - 1].wait()
      return carry

    lax.fori_loop(0, NCHUNK // GB, group, 0)
    plsc.subcore_barrier()

    pltpu.sync_copy(acc.at[pl.ds(r0, ROWS_PT)], out_sums.at[c, pl.ds(r0, ROWS_PT)])

  mesh = plsc.VectorSubcoreMesh(core_axis_name="c", subcore_axis_name="s")
  return pl.kernel(
      body,
      out_type=[jax.ShapeDtypeStruct((NC, ACC_N, D), jnp.float32)],
      mesh=mesh,
      scratch_types=[
          pltpu.VMEM_SHARED((ACC_N, D), jnp.float32),
          pltpu.VMEM((GB, 2, C), jnp.int32),
          pltpu.VMEM((C, D), jnp.float32),
          pltpu.VMEM((C, D), jnp.float32),
          pltpu.SemaphoreType.DMA,
          pltpu.SemaphoreType.DMA,
          pltpu.SemaphoreType.DMA,
          pltpu.SemaphoreType.DMA,
      ])


@functools.cache
def _sc_deg():
  def body(idx5, zfeat, ones_c, out_deg, acc, idxblk, ones_v, ssem):
    c = lax.axis_index("c")
    s = lax.axis_index("s")

    r0 = pl.multiple_of(s * ROWS_PT, 8)
    pltpu.sync_copy(zfeat.at[pl.ds(r0, ROWS_PT)], acc.at[pl.ds(r0, ROWS_PT)])
    pltpu.sync_copy(ones_c, ones_v)
    plsc.subcore_barrier()

    def group(gi, carry):
      pltpu.sync_copy(idx5.at[c, s, pl.ds(gi * GB, GB)], idxblk)
      sd = [pltpu.async_copy(ones_v, acc.at[idxblk.at[k, 1]], ssem, add=True)
            for k in range(GB)]
      for d in sd:
        d.wait()
      return carry

    lax.fori_loop(0, NCHUNK // GB, group, 0)
    plsc.subcore_barrier()

    pltpu.sync_copy(acc.at[pl.ds(r0, ROWS_PT)], out_deg.at[c, pl.ds(r0, ROWS_PT)])

  mesh = plsc.VectorSubcoreMesh(core_axis_name="c", subcore_axis_name="s")
  return pl.kernel(
      body,
      out_type=[jax.ShapeDtypeStruct((NC, ACC_N, D), jnp.float32)],
      mesh=mesh,
      scratch_types=[
          pltpu.VMEM_SHARED((ACC_N, D), jnp.float32),
          pltpu.VMEM((GB, 2, C), jnp.int32),
          pltpu.VMEM((C, D), jnp.float32),
          pltpu.SemaphoreType.DMA,
      ])


_BN = 1000


def _tc_base_body(sp, sn, dp, dn, x, wp, bp, wn, bn, out):
  aggp = sp[...] / jnp.maximum(dp[:, 0:1], 1.0)
  aggn = sn[...] / jnp.maximum(dn[:, 0:1], 1.0)
  xb = x[...]
  out[:, 0:H] = jnp.tanh(aggp @ wp[0:D] + xb @ wp[D:2 * D] + bp[...])
  out[:, H:2 * H] = jnp.tanh(aggn @ wn[0:D] + xb @ wn[D:2 * D] + bn[...])


def _tc_deep_body(sp, sn, dp, dn, hcat, wp, bp, wn, bn, out):
  aggp = sp[...] / jnp.maximum(dp[:, 0:1], 1.0)
  aggn = sn[...] / jnp.maximum(dn[:, 0:1], 1.0)
  hb = hcat[...]
  out[:, 0:H] = jnp.tanh(
      aggp[:, 0:H] @ wp[0:H] + aggn[:, H:2 * H] @ wp[H:2 * H]
      + hb[:, 0:H] @ wp[2 * H:3 * H] + bp[...])
  out[:, H:2 * H] = jnp.tanh(
      aggn[:, 0:H] @ wn[H:2 * H] + aggp[:, H:2 * H] @ wn[0:H]
      + hb[:, H:2 * H] @ wn[2 * H:3 * H] + bn[...])


def _row_block(feat):
  return pl.BlockSpec((_BN, feat), lambda i: (i, 0))


def _full_block(shape):
  return pl.BlockSpec(shape, lambda i: (0,) * len(shape))


def _make_tc(body, kdim):
  return pl.pallas_call(
      body,
      grid=(N // _BN,),
      in_specs=[
          _row_block(D), _row_block(D), _row_block(D), _row_block(D),
          _row_block(D),
          _full_block((kdim, H)), _full_block((1, H)),
          _full_block((kdim, H)), _full_block((1, H)),
      ],
      out_specs=_row_block(D),
      out_shape=jax.ShapeDtypeStruct((N, D), jnp.float32),
  )


_tc_base = _make_tc(_tc_base_body, 2 * D)
_tc_deep = _make_tc(_tc_deep_body, 3 * H)


def kernel(positive_edges, negative_edges, pos_adj, neg_adj, X,
           W_pos_base, b_pos_base, W_neg_base, b_neg_base,
           W_pos_deep, b_pos_deep, W_neg_deep, b_neg_deep):
  del pos_adj, neg_adj
  pad = PAD_E - E
  pad_src = jnp.zeros((1, pad), jnp.int32)
  pad_dst = jnp.full((1, pad), N, jnp.int32)
  srcs = jnp.concatenate([jnp.stack([positive_edges[0], negative_edges[0]]),
                          jnp.broadcast_to(pad_src, (NC, pad))], axis=1)
  dsts = jnp.concatenate([jnp.stack([positive_edges[1], negative_edges[1]]),
                          jnp.broadcast_to(pad_dst, (NC, pad))], axis=1)
  idx5 = jnp.stack([srcs.reshape(NC, NS, NCHUNK, C),
                    dsts.reshape(NC, NS, NCHUNK, C)], axis=3)

  zfeat = jnp.zeros((ACC_N, D), jnp.float32)
  ones_c = jnp.ones((C, D), jnp.float32)

  (deg,) = _sc_deg()(idx5, zfeat, ones_c)
  (sums1,) = _sc_agg()(X, idx5, zfeat)
  hcat = _tc_base(sums1[0], sums1[1], deg[0], deg[1], X,
                  W_pos_base, b_pos_base.reshape(1, H),
                  W_neg_base, b_neg_base.reshape(1, H))
  (sums2,) = _sc_agg()(hcat, idx5, zfeat)
  z = _tc_deep(sums2[0], sums2[1], deg[0], deg[1], hcat,
               W_pos_deep, b_pos_deep.reshape(1, H),
               W_neg_deep, b_neg_deep.reshape(1, H))
  return z
